# Initial kernel scaffold; baseline (speedup 1.0000x reference)
import jax
import jax.numpy as jnp
from jax.experimental import pallas as pl


def kernel(mem, idx, h, W_delta):
    raise NotImplementedError("write your pallas kernel here")


# speedup vs baseline: 1.6194x; 1.6194x over previous
import functools

import jax
import jax.numpy as jnp
from jax import lax
from jax.experimental import pallas as pl
from jax.experimental.pallas import tpu as pltpu
from jax.experimental.pallas import tpu_sc as plsc

NC = 2
NS = 16
NW = NC * NS
LANES = 16

CAPW = 1024
GROUP = 128
CHUNK = 224
NBUF = 3


def _sc_gather(mem, idx):
  B = idx.shape[0]
  M, D = mem.shape
  bw = B // NW
  mesh = plsc.VectorSubcoreMesh(core_axis_name="c", subcore_axis_name="s")

  @functools.partial(
      pl.kernel,
      out_type=jax.ShapeDtypeStruct((B, D), jnp.float32),
      mesh=mesh,
      compiler_params=pltpu.CompilerParams(needs_layout_passes=False),
      scratch_types=[
          pltpu.VMEM((bw,), jnp.int32),
          pltpu.VMEM((bw, D), jnp.float32),
          pltpu.SemaphoreType.DMA,
      ],
  )
  def k(mem_hbm, idx_hbm, out_hbm, idx_v, rows_v, sem):
    wid = lax.axis_index("s") * NC + lax.axis_index("c")
    base = wid * bw
    pltpu.sync_copy(idx_hbm.at[pl.ds(base, bw)], idx_v)
    pltpu.async_copy(mem_hbm.at[idx_v], rows_v, sem).wait()
    pltpu.sync_copy(rows_v, out_hbm.at[pl.ds(base, bw)])

  return k(mem, idx)


def _tc_update(e, h, w):
  B, D = e.shape
  blk = 2048

  def body(e_ref, h_ref, w_ref, o_ref):
    eb = e_ref[...]
    hb = h_ref[...]
    v = lax.dot_general(hb, w_ref[...], (((1,), (1,)), ((), ())),
                        preferred_element_type=jnp.float32)
    bil = jnp.sum(eb * v, axis=1, keepdims=True)
    delta = jax.nn.sigmoid(bil)
    upd = delta * eb + (1.0 - delta) * hb
    nrm = jnp.sqrt(jnp.sum(upd * upd, axis=1, keepdims=True))
    o_ref[...] = upd / (nrm + 1e-12)

  return pl.pallas_call(
      body,
      grid=(B // blk,),
      in_specs=[
          pl.BlockSpec((blk, D), lambda i: (i, 0)),
          pl.BlockSpec((blk, D), lambda i: (i, 0)),
          pl.BlockSpec((D, D), lambda i: (0, 0)),
      ],
      out_specs=pl.BlockSpec((blk, D), lambda i: (i, 0)),
      out_shape=jax.ShapeDtypeStruct((B, D), jnp.float32),
  )(e, h, w)


def _sc_scatter(mem, idx, upd):
  M, D = mem.shape
  B = idx.shape[0]
  rw_a = ((M + NW - 1) // NW + 7) // 8 * 8
  rw_last = M - (NW - 1) * rw_a
  assert 0 < rw_last <= rw_a and rw_last % 8 == 0 and M % 8 == 0
  nvec = B // LANES
  grp_rows = CAPW // GROUP
  nfc = rw_last // CHUNK
  tail_a = rw_a - nfc * CHUNK
  tail_l = rw_last - nfc * CHUNK
  assert tail_a % 8 == 0 and tail_l % 8 == 0 and nfc >= 2
  piece = -(-nvec // nfc)
  mesh = plsc.VectorSubcoreMesh(core_axis_name="c", subcore_axis_name="s")

  @functools.partial(
      pl.kernel,
      out_type=jax.ShapeDtypeStruct((M, D), jnp.float32),
      mesh=mesh,
      compiler_params=pltpu.CompilerParams(needs_layout_passes=False),
      scratch_types=[
          pltpu.VMEM((B,), jnp.int32),
          pltpu.VMEM((rw_a,), jnp.int32),
          pltpu.VMEM((CAPW + LANES,), jnp.int32),
          pltpu.VMEM((CAPW + LANES,), jnp.int32),
          pltpu.VMEM((CAPW,), jnp.int32),
          pltpu.VMEM((CAPW,), jnp.int32),
          pltpu.VMEM((grp_rows, GROUP), jnp.int32),
          pltpu.VMEM((grp_rows, GROUP), jnp.int32),
          pltpu.VMEM((GROUP, D), jnp.float32),
          pltpu.VMEM_SHARED((NS * NBUF * CHUNK, D), jnp.float32),
          pltpu.SemaphoreType.DMA,
          pltpu.SemaphoreType.DMA,
          pltpu.SemaphoreType.DMA,
          pltpu.SemaphoreType.DMA,
          pltpu.SemaphoreType.DMA,
          pltpu.SemaphoreType.DMA,
          pltpu.SemaphoreType.DMA,
          pltpu.SemaphoreType.DMA,
          pltpu.SemaphoreType.DMA,
      ],
  )
  def k(mem_hbm, idx_hbm, upd_hbm, out_hbm, idx_v, mark, sel_d, sel_b,
        wd1, wb1, wdest, wb, grows, spb, gsem, ssem, isem,
        rsem0, rsem1, rsem2, wsem0, wsem1, wsem2):
    sid = lax.axis_index("s")
    wid = sid * NC + lax.axis_index("c")
    lo = wid * rw_a
    hi = jnp.minimum(lo + rw_a, M)

    sbase = sid * (NBUF * CHUNK)
    bufs = tuple(spb.at[pl.ds(sbase + b * CHUNK, CHUNK)] for b in range(NBUF))

    idx_cp = pltpu.async_copy(idx_hbm, idx_v, isem)
    rsems = (rsem0, rsem1, rsem2)
    pend_r = [
        pltpu.async_copy(mem_hbm.at[pl.ds(lo, CHUNK)], bufs[0], rsem0),
        pltpu.async_copy(mem_hbm.at[pl.ds(lo + CHUNK, CHUNK)], bufs[1], rsem1),
        None,
    ]
    idx_cp.wait()
    lanes = lax.iota(jnp.int32, LANES)
    lane0 = lanes == 0
    lov = jnp.full((LANES,), lo, jnp.int32)
    hiv = jnp.full((LANES,), hi, jnp.int32)

    def scan_body(i, cnt):
      v = idx_v[pl.ds(i * LANES, LANES)]
      m = jnp.logical_and(v >= lov, v < hiv)
      pop = jnp.sum(m.astype(jnp.int32))
      off = jnp.minimum(cnt, CAPW)
      plsc.store_compressed(sel_d.at[pl.ds(off, LANES)], v, mask=m)
      plsc.store_compressed(sel_b.at[pl.ds(off, LANES)], i * LANES + lanes, mask=m)
      return jnp.minimum(cnt + pop, CAPW)

    wsems = (wsem0, wsem1, wsem2)
    pend_w = [None, None, None]
    cnt = jnp.int32(0)
    for kk in range(nfc):
      b = kk % NBUF
      s, e = kk * piece, min((kk + 1) * piece, nvec)
      if s < e:
        cnt = lax.fori_loop(s, e, scan_body, cnt, unroll=False)
      pend_r[b].wait()
      pend_w[b] = pltpu.async_copy(
          bufs[b], out_hbm.at[pl.ds(lo + kk * CHUNK, CHUNK)], wsems[b])
      j = kk + 2
      if j < nfc:
        jb = j % NBUF
        if pend_w[jb] is not None:
          pend_w[jb].wait()
          pend_w[jb] = None
        pend_r[jb] = pltpu.async_copy(
            mem_hbm.at[pl.ds(lo + j * CHUNK, CHUNK)], bufs[jb], rsems[jb])
    for b in range(NBUF):
      if pend_w[b] is not None:
        pend_w[b].wait()
    if tail_a:
      @pl.when(wid < NW - 1)
      def _tail_a():
        tb = spb.at[pl.ds(sbase, tail_a)]
        pltpu.sync_copy(mem_hbm.at[pl.ds(lo + nfc * CHUNK, tail_a)], tb)
        pltpu.sync_copy(tb, out_hbm.at[pl.ds(lo + nfc * CHUNK, tail_a)])
    if tail_l:
      @pl.when(wid == NW - 1)
      def _tail_l():
        tb = spb.at[pl.ds(sbase, tail_l)]
        pltpu.sync_copy(mem_hbm.at[pl.ds(lo + nfc * CHUNK, tail_l)], tb)
        pltpu.sync_copy(tb, out_hbm.at[pl.ds(lo + nfc * CHUNK, tail_l)])

    nchunk = (cnt + LANES - 1) // LANES

    def mark_body(t, carry):
      dv = jnp.clip(sel_d[pl.ds(t * LANES, LANES)] - lo, 0, rw_a - 1)
      base = t * LANES
      for l in range(LANES):
        m = jnp.logical_and(lane0, base + l < cnt)
        plsc.store_scatter(mark, [jnp.full((LANES,), dv[l], jnp.int32)],
                           jnp.full((LANES,), base + l, jnp.int32), mask=m)
      return carry

    lax.fori_loop(0, nchunk, mark_body, jnp.int32(0), unroll=False)

    def win_body(t, wcnt):
      base = t * LANES
      dv = sel_d[pl.ds(base, LANES)]
      bv = sel_b[pl.ds(base, LANES)]
      mk = plsc.load_gather(mark, [jnp.clip(dv - lo, 0, rw_a - 1)])
      isw = jnp.logical_and(mk == base + lanes, base + lanes < cnt)
      plsc.store_compressed(wd1.at[pl.ds(wcnt, LANES)], dv, mask=isw)
      plsc.store_compressed(wb1.at[pl.ds(wcnt, LANES)], bv, mask=isw)
      return wcnt + jnp.sum(isw.astype(jnp.int32))

    wcnt = lax.fori_loop(0, nchunk, win_body, jnp.int32(0), unroll=False)

    w0 = wd1[pl.ds(0, LANES)][0]
    b0 = wb1[pl.ds(0, LANES)][0]
    for kk in range(CAPW // LANES):
      r = kk // (GROUP // LANES)
      c = (kk % (GROUP // LANES)) * LANES
      keep = (kk * LANES + lanes) < wcnt
      wdest[r, pl.ds(c, LANES)] = jnp.where(keep, wd1[pl.ds(kk * LANES, LANES)], w0)
      wb[r, pl.ds(c, LANES)] = jnp.where(keep, wb1[pl.ds(kk * LANES, LANES)], b0)

    ngrp = (wcnt + (GROUP - 1)) // GROUP

    def grp_body(g, carry):
      pltpu.async_copy(upd_hbm.at[wb.at[g]], grows, gsem).wait()
      pltpu.async_copy(grows, out_hbm.at[wdest.at[g]], ssem).wait()
      return carry

    lax.fori_loop(0, ngrp, grp_body, jnp.int32(0), unroll=False)

  return k(mem, idx, upd)


def kernel(mem, idx, h, W_delta):
  e = _sc_gather(mem, idx)
  upd = _tc_update(e, h, W_delta)
  return _sc_scatter(mem, idx, upd)


# speedup vs baseline: 1.6386x; 1.0119x over previous
import functools

import jax
import jax.numpy as jnp
from jax import lax
from jax.experimental import pallas as pl
from jax.experimental.pallas import tpu as pltpu
from jax.experimental.pallas import tpu_sc as plsc

NC = 2
NS = 16
NW = NC * NS
LANES = 16

CAPW = 1024
GROUP = 128
CHUNK = 184
NBUF = 3


def _sc_gather(mem, idx):
  B = idx.shape[0]
  M, D = mem.shape
  bw = B // NW
  mesh = plsc.VectorSubcoreMesh(core_axis_name="c", subcore_axis_name="s")

  @functools.partial(
      pl.kernel,
      out_type=jax.ShapeDtypeStruct((B, D), jnp.float32),
      mesh=mesh,
      compiler_params=pltpu.CompilerParams(needs_layout_passes=False),
      scratch_types=[
          pltpu.VMEM((bw,), jnp.int32),
          pltpu.VMEM((bw, D), jnp.float32),
          pltpu.SemaphoreType.DMA,
      ],
  )
  def k(mem_hbm, idx_hbm, out_hbm, idx_v, rows_v, sem):
    wid = lax.axis_index("s") * NC + lax.axis_index("c")
    base = wid * bw
    pltpu.sync_copy(idx_hbm.at[pl.ds(base, bw)], idx_v)
    pltpu.async_copy(mem_hbm.at[idx_v], rows_v, sem).wait()
    pltpu.sync_copy(rows_v, out_hbm.at[pl.ds(base, bw)])

  return k(mem, idx)


def _tc_update(e, h, w):
  B, D = e.shape
  blk = 2048

  def body(e_ref, h_ref, w_ref, o_ref):
    eb = e_ref[...]
    hb = h_ref[...]
    v = lax.dot_general(hb, w_ref[...], (((1,), (1,)), ((), ())),
                        preferred_element_type=jnp.float32)
    bil = jnp.sum(eb * v, axis=1, keepdims=True)
    delta = jax.nn.sigmoid(bil)
    upd = delta * eb + (1.0 - delta) * hb
    nrm = jnp.sqrt(jnp.sum(upd * upd, axis=1, keepdims=True))
    o_ref[...] = upd / (nrm + 1e-12)

  return pl.pallas_call(
      body,
      grid=(B // blk,),
      in_specs=[
          pl.BlockSpec((blk, D), lambda i: (i, 0)),
          pl.BlockSpec((blk, D), lambda i: (i, 0)),
          pl.BlockSpec((D, D), lambda i: (0, 0)),
      ],
      out_specs=pl.BlockSpec((blk, D), lambda i: (i, 0)),
      out_shape=jax.ShapeDtypeStruct((B, D), jnp.float32),
  )(e, h, w)


def _sc_scatter(mem, idx, upd):
  M, D = mem.shape
  B = idx.shape[0]
  rw_a = ((M + NW - 1) // NW + 7) // 8 * 8
  rw_last = M - (NW - 1) * rw_a
  assert 0 < rw_last <= rw_a and rw_last % 8 == 0 and M % 8 == 0
  nvec = B // LANES
  grp_rows = CAPW // GROUP
  nfc = rw_last // CHUNK
  tail_a = rw_a - nfc * CHUNK
  tail_l = rw_last - nfc * CHUNK
  assert tail_a % 8 == 0 and tail_l % 8 == 0 and nfc >= 2
  piece = -(-nvec // nfc)
  mesh = plsc.VectorSubcoreMesh(core_axis_name="c", subcore_axis_name="s")

  @functools.partial(
      pl.kernel,
      out_type=jax.ShapeDtypeStruct((M, D), jnp.float32),
      mesh=mesh,
      compiler_params=pltpu.CompilerParams(needs_layout_passes=False),
      scratch_types=[
          pltpu.VMEM((B,), jnp.int32),
          pltpu.VMEM((rw_a,), jnp.int32),
          pltpu.VMEM((CAPW + LANES,), jnp.int32),
          pltpu.VMEM((CAPW + LANES,), jnp.int32),
          pltpu.VMEM((CAPW,), jnp.int32),
          pltpu.VMEM((CAPW,), jnp.int32),
          pltpu.VMEM((grp_rows, GROUP), jnp.int32),
          pltpu.VMEM((grp_rows, GROUP), jnp.int32),
          pltpu.VMEM((GROUP, D), jnp.float32),
          pltpu.VMEM((GROUP, D), jnp.float32),
          pltpu.VMEM_SHARED((NS * NBUF * CHUNK, D), jnp.float32),
          pltpu.SemaphoreType.DMA,
          pltpu.SemaphoreType.DMA,
          pltpu.SemaphoreType.DMA,
          pltpu.SemaphoreType.DMA,
          pltpu.SemaphoreType.DMA,
          pltpu.SemaphoreType.DMA,
          pltpu.SemaphoreType.DMA,
          pltpu.SemaphoreType.DMA,
          pltpu.SemaphoreType.DMA,
      ],
  )
  def k(mem_hbm, idx_hbm, upd_hbm, out_hbm, idx_v, mark, sel_d, sel_b,
        wd1, wb1, wdest, wb, growsa, growsb, spb, gsem, ssem, isem,
        rsem0, rsem1, rsem2, wsem0, wsem1, wsem2):
    sid = lax.axis_index("s")
    wid = sid * NC + lax.axis_index("c")
    lo = wid * rw_a
    hi = jnp.minimum(lo + rw_a, M)

    sbase = sid * (NBUF * CHUNK)
    bufs = tuple(spb.at[pl.ds(sbase + b * CHUNK, CHUNK)] for b in range(NBUF))

    idx_cp = pltpu.async_copy(idx_hbm, idx_v, isem)
    rsems = (rsem0, rsem1, rsem2)
    pend_r = [
        pltpu.async_copy(mem_hbm.at[pl.ds(lo, CHUNK)], bufs[0], rsem0),
        pltpu.async_copy(mem_hbm.at[pl.ds(lo + CHUNK, CHUNK)], bufs[1], rsem1),
        None,
    ]
    idx_cp.wait()
    lanes = lax.iota(jnp.int32, LANES)
    lane0 = lanes == 0
    lov = jnp.full((LANES,), lo, jnp.int32)
    hiv = jnp.full((LANES,), hi, jnp.int32)

    def scan_body(i, cnt):
      v = idx_v[pl.ds(i * LANES, LANES)]
      m = jnp.logical_and(v >= lov, v < hiv)
      pop = jnp.sum(m.astype(jnp.int32))
      off = jnp.minimum(cnt, CAPW)
      plsc.store_compressed(sel_d.at[pl.ds(off, LANES)], v, mask=m)
      plsc.store_compressed(sel_b.at[pl.ds(off, LANES)], i * LANES + lanes, mask=m)
      return jnp.minimum(cnt + pop, CAPW)

    wsems = (wsem0, wsem1, wsem2)
    pend_w = [None, None, None]
    cnt = jnp.int32(0)
    for kk in range(nfc):
      b = kk % NBUF
      s, e = kk * piece, min((kk + 1) * piece, nvec)
      if s < e:
        cnt = lax.fori_loop(s, e, scan_body, cnt, unroll=False)
      pend_r[b].wait()
      pend_w[b] = pltpu.async_copy(
          bufs[b], out_hbm.at[pl.ds(lo + kk * CHUNK, CHUNK)], wsems[b])
      j = kk + 2
      if j < nfc:
        jb = j % NBUF
        if pend_w[jb] is not None:
          pend_w[jb].wait()
          pend_w[jb] = None
        pend_r[jb] = pltpu.async_copy(
            mem_hbm.at[pl.ds(lo + j * CHUNK, CHUNK)], bufs[jb], rsems[jb])
    for b in range(NBUF):
      if pend_w[b] is not None:
        pend_w[b].wait()
    if tail_a:
      @pl.when(wid < NW - 1)
      def _tail_a():
        tb = spb.at[pl.ds(sbase, tail_a)]
        pltpu.sync_copy(mem_hbm.at[pl.ds(lo + nfc * CHUNK, tail_a)], tb)
        pltpu.sync_copy(tb, out_hbm.at[pl.ds(lo + nfc * CHUNK, tail_a)])
    if tail_l:
      @pl.when(wid == NW - 1)
      def _tail_l():
        tb = spb.at[pl.ds(sbase, tail_l)]
        pltpu.sync_copy(mem_hbm.at[pl.ds(lo + nfc * CHUNK, tail_l)], tb)
        pltpu.sync_copy(tb, out_hbm.at[pl.ds(lo + nfc * CHUNK, tail_l)])

    nchunk = (cnt + LANES - 1) // LANES

    def mark_body(t, carry):
      dv = jnp.clip(sel_d[pl.ds(t * LANES, LANES)] - lo, 0, rw_a - 1)
      base = t * LANES
      for l in range(LANES):
        m = jnp.logical_and(lane0, base + l < cnt)
        plsc.store_scatter(mark, [jnp.full((LANES,), dv[l], jnp.int32)],
                           jnp.full((LANES,), base + l, jnp.int32), mask=m)
      return carry

    lax.fori_loop(0, nchunk, mark_body, jnp.int32(0), unroll=False)

    def win_body(t, wcnt):
      base = t * LANES
      dv = sel_d[pl.ds(base, LANES)]
      bv = sel_b[pl.ds(base, LANES)]
      mk = plsc.load_gather(mark, [jnp.clip(dv - lo, 0, rw_a - 1)])
      isw = jnp.logical_and(mk == base + lanes, base + lanes < cnt)
      plsc.store_compressed(wd1.at[pl.ds(wcnt, LANES)], dv, mask=isw)
      plsc.store_compressed(wb1.at[pl.ds(wcnt, LANES)], bv, mask=isw)
      return wcnt + jnp.sum(isw.astype(jnp.int32))

    wcnt = lax.fori_loop(0, nchunk, win_body, jnp.int32(0), unroll=False)

    w0 = wd1[pl.ds(0, LANES)][0]
    b0 = wb1[pl.ds(0, LANES)][0]
    for kk in range(CAPW // LANES):
      r = kk // (GROUP // LANES)
      c = (kk % (GROUP // LANES)) * LANES
      keep = (kk * LANES + lanes) < wcnt
      wdest[r, pl.ds(c, LANES)] = jnp.where(keep, wd1[pl.ds(kk * LANES, LANES)], w0)
      wb[r, pl.ds(c, LANES)] = jnp.where(keep, wb1[pl.ds(kk * LANES, LANES)], b0)

    ngrp = (wcnt + (GROUP - 1)) // GROUP
    npair = (ngrp + 1) // 2

    def pair_body(p, carry):
      ga = pltpu.async_copy(upd_hbm.at[wb.at[2 * p]], growsa, gsem)
      gb = pltpu.async_copy(upd_hbm.at[wb.at[2 * p + 1]], growsb, isem)
      ga.wait()
      sa = pltpu.async_copy(growsa, out_hbm.at[wdest.at[2 * p]], ssem)
      gb.wait()
      sb = pltpu.async_copy(growsb, out_hbm.at[wdest.at[2 * p + 1]], rsem0)
      sa.wait()
      sb.wait()
      return carry

    lax.fori_loop(0, npair, pair_body, jnp.int32(0), unroll=False)

  return k(mem, idx, upd)


def kernel(mem, idx, h, W_delta):
  e = _sc_gather(mem, idx)
  upd = _tc_update(e, h, W_delta)
  return _sc_scatter(mem, idx, upd)
